# Initial kernel scaffold; baseline (speedup 1.0000x reference)
import jax
import jax.numpy as jnp
from jax.experimental import pallas as pl


def kernel(x, type_embeddings, atom_types, W_gate, W_experts, b_experts):
    raise NotImplementedError("write your pallas kernel here")


# speedup vs baseline: 1.4169x; 1.4169x over previous
import functools

import jax
import jax.numpy as jnp
from jax.experimental import pallas as pl
from jax.experimental.pallas import tpu as pltpu


def _dense_moe_body(pew_ref, x_ref, types_ref, w_ref, b_ref, out_ref, *, n_experts, n_types):
    tm = x_ref.shape[0]
    x = x_ref[...]
    types = types_ref[...]
    lanes = jax.lax.broadcasted_iota(jnp.int32, (tm, n_types), 1)
    onehot = (types == lanes).astype(jnp.bfloat16)
    wt = jnp.dot(onehot, pew_ref[...], preferred_element_type=jnp.float32)
    acc = jnp.zeros(out_ref.shape, jnp.float32)
    for e in range(n_experts):
        y = jnp.dot(x, w_ref[e], preferred_element_type=jnp.float32)
        y = jnp.tanh(y + b_ref[e][None, :])
        acc = acc + wt[:, e][:, None] * y
    out_ref[...] = acc


def kernel(x, type_embeddings, atom_types, W_gate, W_experts, b_experts):
    nb, nloc, d_in = x.shape
    n_types, _ = type_embeddings.shape
    n_experts, _, d_out = W_experts.shape
    top_k = 2
    n = nb * nloc

    logits = type_embeddings @ W_gate
    topk_logits, topk_idx = jax.lax.top_k(logits, top_k)
    w = jax.nn.softmax(topk_logits, axis=-1)
    pew = jnp.zeros((n_types, n_experts), jnp.float32)
    pew = pew.at[jnp.arange(n_types)[:, None], topk_idx].add(w)

    x_flat = x.reshape(n, d_in).astype(jnp.bfloat16)
    types2d = atom_types.reshape(n, 1).astype(jnp.int32)
    w_bf = W_experts.astype(jnp.bfloat16)
    pew_bf = pew.astype(jnp.bfloat16)

    tm = 512
    grid = (n // tm,)
    out = pl.pallas_call(
        functools.partial(_dense_moe_body, n_experts=n_experts, n_types=n_types),
        grid=grid,
        in_specs=[
            pl.BlockSpec((n_types, n_experts), lambda i: (0, 0)),
            pl.BlockSpec((tm, d_in), lambda i: (i, 0)),
            pl.BlockSpec((tm, 1), lambda i: (i, 0)),
            pl.BlockSpec((n_experts, d_in, d_out), lambda i: (0, 0, 0)),
            pl.BlockSpec((n_experts, d_out), lambda i: (0, 0)),
        ],
        out_specs=pl.BlockSpec((tm, d_out), lambda i: (i, 0)),
        out_shape=jax.ShapeDtypeStruct((n, d_out), jnp.float32),
        compiler_params=pltpu.CompilerParams(
            dimension_semantics=("arbitrary",),
        ),
    )(pew_bf, x_flat, types2d, w_bf, b_experts)
    return out.reshape(nb, nloc, d_out)
